# baseline (speedup 1.0000x reference)
import jax
import jax.numpy as jnp
from jax import lax
from jax.experimental import pallas as pl
from jax.experimental.pallas import tpu as pltpu
from jax.experimental.pallas import tpu_sc as plsc

_B = 16384
_D = 64
_REL = 5
_L = 16
_NC = 2
_NS = 16
_NW = _NC * _NS
_BPW = _B // _NW
_CH = 128
_NCH = _BPW // _CH
_NE = 1000000
_BAND = 128
_NFULL = _NE // _BAND
_TAIL0 = _NFULL * _BAND
_NIDS = 2 * _B
_NST = _NFULL + 2
_RING = 32

_mesh = plsc.VectorSubcoreMesh(core_axis_name="c", subcore_axis_name="s",
                               num_cores=_NC, num_subcores=_NS)
_params = pltpu.CompilerParams(needs_layout_passes=False)


def _vsqrt(x):
    i = plsc.bitcast(x, jnp.int32)
    i = jnp.int32(0x5F3759DF) - lax.shift_right_logical(i, 1)
    y = plsc.bitcast(i, jnp.float32)
    for _ in range(3):
        y = y * (1.5 - 0.5 * x * y * y)
    return x * y


def _remap_body(hidx_hbm, tidx_hbm, nid_hbm, ids_hbm,
                hi_v, ti_v, hm_v, tm_v, sem0, sem1):
    wid = lax.axis_index("s") * _NC + lax.axis_index("c")
    base = wid * _BPW
    for c in range(_NCH):
        off = base + c * _CH
        pltpu.sync_copy(hidx_hbm.at[pl.ds(off, _CH)], hi_v)
        pltpu.sync_copy(tidx_hbm.at[pl.ds(off, _CH)], ti_v)
        cp_h = pltpu.async_copy(nid_hbm.at[hi_v], hm_v, sem0)
        cp_t = pltpu.async_copy(nid_hbm.at[ti_v], tm_v, sem1)
        cp_h.wait()
        cp_t.wait()
        pltpu.sync_copy(hm_v, ids_hbm.at[pl.ds(off, _CH)])
        pltpu.sync_copy(tm_v, ids_hbm.at[pl.ds(_B + off, _CH)])


_sc_remap = pl.kernel(
    _remap_body,
    out_type=jax.ShapeDtypeStruct((_NIDS,), jnp.int32),
    mesh=_mesh,
    compiler_params=_params,
    scratch_types=[
        pltpu.VMEM((_CH,), jnp.int32),
        pltpu.VMEM((_CH,), jnp.int32),
        pltpu.VMEM((_CH,), jnp.int32),
        pltpu.VMEM((_CH,), jnp.int32),
        pltpu.SemaphoreType.DMA,
        pltpu.SemaphoreType.DMA,
    ],
)


def _gather_body(sids_hbm, perm_hbm, starts_hbm, entt_hbm, tail_hbm, rel_hbm,
                 rows_hbm, ids_v, perm_v, starts_v, band_v, tail_v, rel_v,
                 ring_v, semb0, semb1, semr):
    wid = lax.axis_index("s") * _NC + lax.axis_index("c")
    pltpu.sync_copy(sids_hbm, ids_v)
    pltpu.sync_copy(perm_hbm, perm_v)
    pltpu.sync_copy(starts_hbm, starts_v)
    pltpu.sync_copy(tail_hbm, tail_v)
    pltpu.sync_copy(rel_hbm, rel_v)
    lanes = lax.iota(jnp.int32, _L)
    rel_c = [rel_v[pl.ds(kk * _L, _L)] for kk in range(4)]
    nb = (_NFULL - 1 - wid) // _NW + 1
    nb2 = (nb + 1) // 2

    def clamped_band(k):
        return jnp.minimum(wid + k * _NW, _NFULL - 1)

    def fire_band(k, par, sem):
        b0 = pl.multiple_of(clamped_band(k) * _BAND, _BAND)
        for tc in range(8):
            pltpu.async_copy(
                entt_hbm.at[pl.ds(tc * 8, 8), pl.ds(b0, _BAND)],
                band_v.at[pl.ds(par * _D + tc * 8, 8)], sem)

    def wait_band(par, sem):
        pltpu.make_async_copy(entt_hbm.at[pl.ds(0, _D), pl.ds(0, _BAND)],
                              band_v.at[pl.ds(par * _D, _D)], sem).wait()

    def splat(x):
        return jnp.full((_L,), x, jnp.int32)

    def emit_positions(s, e, cnt, src_ref, row_off, tail_mode):
        def pos_body(p, cnt):
            e_id = plsc.load_gather(ids_v, [splat(p)])[0]
            pm = plsc.load_gather(perm_v, [splat(p)])[0]
            col = splat(e_id - _TAIL0 if tail_mode else e_id & (_BAND - 1))
            is_h = pm < _B
            slot = cnt % _RING
            for kk in range(4):
                v = plsc.load_gather(src_ref,
                                     [row_off + lanes + kk * _L, col])
                v = jnp.where(is_h, v + rel_c[kk], v)
                ring_v[pl.ds(slot * _D + kk * _L, _L)] = v
            pltpu.async_copy(ring_v.at[pl.ds(slot * _D, _D)],
                             rows_hbm.at[pl.ds(pm * _D, _D)], semr)
            cnt = cnt + 1

            @pl.when(cnt % _RING == 0)
            def _():
                pltpu.make_async_copy(rows_hbm.at[pl.ds(0, _RING * _D)],
                                      ring_v, semr).wait()

            return cnt

        return lax.fori_loop(s, e, pos_body, cnt)

    def starts_at(i):
        return plsc.load_gather(starts_v, [splat(i)])[0]

    def process_band(k, cnt, par):
        band = clamped_band(k)
        s = starts_at(band)
        e = starts_at(band + 1)
        return emit_positions(s, e, cnt, band_v, par * _D, False)

    fire_band(0, 0, semb0)

    def pair_body(k2, cnt):
        k = 2 * k2
        wait_band(0, semb0)
        fire_band(k + 1, 1, semb1)
        cnt = process_band(k, cnt, 0)
        wait_band(1, semb1)
        fire_band(k + 2, 0, semb0)
        cnt = process_band(k + 1, cnt, 1)
        return cnt

    cnt = lax.fori_loop(0, nb2, pair_body, jnp.int32(0))
    wait_band(0, semb0)

    tail_s = jnp.where(wid == _NW - 1, starts_at(_NFULL), _NIDS)
    cnt = emit_positions(tail_s, jnp.int32(_NIDS), cnt, tail_v, 0, True)

    def drain_body(i, x):
        pltpu.make_async_copy(rows_hbm.at[pl.ds(0, _D)],
                              ring_v.at[pl.ds(0, _D)], semr).wait()
        return x

    lax.fori_loop(0, cnt % _RING, drain_body, 0)


_sc_gather = pl.kernel(
    _gather_body,
    out_type=jax.ShapeDtypeStruct((_NIDS * _D,), jnp.float32),
    mesh=_mesh,
    compiler_params=_params,
    scratch_types=[
        pltpu.VMEM((_NIDS,), jnp.int32),
        pltpu.VMEM((_NIDS,), jnp.int32),
        pltpu.VMEM((_NST,), jnp.int32),
        pltpu.VMEM((2 * _D, _BAND), jnp.float32),
        pltpu.VMEM((_D, _D), jnp.float32),
        pltpu.VMEM((_D,), jnp.float32),
        pltpu.VMEM((_RING * _D,), jnp.float32),
        pltpu.SemaphoreType.DMA,
        pltpu.SemaphoreType.DMA,
        pltpu.SemaphoreType.DMA,
    ],
)


def _score_body(rows_hbm, out_hbm, hrow_v, trow_v, out_v, sem0, sem1):
    wid = lax.axis_index("s") * _NC + lax.axis_index("c")
    base = wid * _BPW
    lanes = lax.iota(jnp.int32, _L)
    for c in range(_NCH):
        off = base + c * _CH
        cp_h = pltpu.async_copy(rows_hbm.at[pl.ds(off * _D, _CH * _D)],
                                hrow_v, sem0)
        cp_t = pltpu.async_copy(
            rows_hbm.at[pl.ds((_B + off) * _D, _CH * _D)], trow_v, sem1)
        cp_h.wait()
        cp_t.wait()

        def group_body(g, carry):
            rows = (lanes + g * _L) * _D

            def dim_body(d, acc):
                idx = rows + d
                hd = plsc.load_gather(hrow_v, [idx])
                td = plsc.load_gather(trow_v, [idx])
                diff = hd - td
                return acc + diff * diff

            acc = lax.fori_loop(0, _D, dim_body,
                                jnp.zeros((_L,), jnp.float32))
            out_v[pl.ds(c * _CH + g * _L, _L)] = _vsqrt(acc + 1e-12)
            return carry

        lax.fori_loop(0, _CH // _L, group_body, 0)
    pltpu.sync_copy(out_v, out_hbm.at[pl.ds(base, _BPW)])


_sc_score = pl.kernel(
    _score_body,
    out_type=jax.ShapeDtypeStruct((_B,), jnp.float32),
    mesh=_mesh,
    compiler_params=_params,
    scratch_types=[
        pltpu.VMEM((_CH * _D,), jnp.float32),
        pltpu.VMEM((_CH * _D,), jnp.float32),
        pltpu.VMEM((_BPW,), jnp.float32),
        pltpu.SemaphoreType.DMA,
        pltpu.SemaphoreType.DMA,
    ],
)


@jax.jit
def kernel(data, node_ids, entity_emb, relation_emb):
    hidx = data[:, 0]
    tidx = data[:, 1]
    ids = _sc_remap(hidx, tidx, node_ids)
    iota = lax.iota(jnp.int32, _NIDS)
    sids, perm = lax.sort_key_val(ids, iota)
    bounds = jnp.arange(_NST, dtype=jnp.int32) * _BAND
    starts = jnp.searchsorted(sids, bounds).astype(jnp.int32)
    entt = entity_emb.T
    tail = entt[:, _TAIL0:]
    rel_row = relation_emb[_REL]
    rows = _sc_gather(sids, perm, starts, entt, tail, rel_row)
    out = _sc_score(rows)
    return out[:, None]


# speedup vs baseline: 4.4301x; 4.4301x over previous
import jax
import jax.numpy as jnp
from jax import lax
from jax.experimental import pallas as pl
from jax.experimental.pallas import tpu as pltpu
from jax.experimental.pallas import tpu_sc as plsc

_B = 16384
_D = 64
_REL = 5
_L = 16
_NC = 2
_NS = 16
_NW = _NC * _NS
_BPW = _B // _NW
_CH = 128
_NCH = _BPW // _CH
_NE = 1000000
_BAND = 128
_NFULL = _NE // _BAND
_TAIL0 = _NFULL * _BAND
_NIDS = 2 * _B
_NST = _NFULL + 2
_RING = 32

_mesh = plsc.VectorSubcoreMesh(core_axis_name="c", subcore_axis_name="s",
                               num_cores=_NC, num_subcores=_NS)
_params = pltpu.CompilerParams(needs_layout_passes=False)


def _vsqrt(x):
    i = plsc.bitcast(x, jnp.int32)
    i = jnp.int32(0x5F3759DF) - lax.shift_right_logical(i, 1)
    y = plsc.bitcast(i, jnp.float32)
    for _ in range(3):
        y = y * (1.5 - 0.5 * x * y * y)
    return x * y


def _remap_body(hidx_hbm, tidx_hbm, nid_hbm, ids_hbm,
                hi_v, ti_v, hm_v, tm_v, sem0, sem1):
    wid = lax.axis_index("s") * _NC + lax.axis_index("c")
    base = wid * _BPW
    for c in range(_NCH):
        off = base + c * _CH
        pltpu.sync_copy(hidx_hbm.at[pl.ds(off, _CH)], hi_v)
        pltpu.sync_copy(tidx_hbm.at[pl.ds(off, _CH)], ti_v)
        cp_h = pltpu.async_copy(nid_hbm.at[hi_v], hm_v, sem0)
        cp_t = pltpu.async_copy(nid_hbm.at[ti_v], tm_v, sem1)
        cp_h.wait()
        cp_t.wait()
        pltpu.sync_copy(hm_v, ids_hbm.at[pl.ds(off, _CH)])
        pltpu.sync_copy(tm_v, ids_hbm.at[pl.ds(_B + off, _CH)])


_sc_remap = pl.kernel(
    _remap_body,
    out_type=jax.ShapeDtypeStruct((_NIDS,), jnp.int32),
    mesh=_mesh,
    compiler_params=_params,
    scratch_types=[
        pltpu.VMEM((_CH,), jnp.int32),
        pltpu.VMEM((_CH,), jnp.int32),
        pltpu.VMEM((_CH,), jnp.int32),
        pltpu.VMEM((_CH,), jnp.int32),
        pltpu.SemaphoreType.DMA,
        pltpu.SemaphoreType.DMA,
    ],
)


def _gather_body(sids_hbm, perm_hbm, entt_hbm, tail_hbm, rel_hbm,
                 rows_hbm, ids_v, perm_v, sloc_v, eloc_v, band_v, tail_v,
                 rel_v, ring_v, semb0, semb1, semb2, semb3, semr):
    wid = lax.axis_index("s") * _NC + lax.axis_index("c")
    pltpu.sync_copy(sids_hbm, ids_v)
    pltpu.sync_copy(perm_hbm, perm_v)
    pltpu.sync_copy(tail_hbm, tail_v)
    pltpu.sync_copy(rel_hbm, rel_v)
    lanes = lax.iota(jnp.int32, _L)
    rel_c = [rel_v[pl.ds(kk * _L, _L)] for kk in range(4)]
    sems = [semb0, semb1, semb2, semb3]
    nb = (_NFULL - 1 - wid) // _NW + 1
    nb4 = (nb + 3) // 4

    def bsearch(tgt):
        lo = jnp.zeros((_L,), jnp.int32)
        hi = jnp.full((_L,), _NIDS, jnp.int32)
        for _ in range(16):
            act = lo < hi
            mid = lax.shift_right_logical(lo + hi, 1)
            v = plsc.load_gather(ids_v,
                                 [jnp.minimum(mid, _NIDS - 1)])
            right = jnp.logical_and(act, v < tgt)
            lo = jnp.where(right, mid + 1, lo)
            hi = jnp.where(jnp.logical_and(act, jnp.logical_not(right)),
                           mid, hi)
        return lo

    for grp in range(16):
        bvec = wid + (lanes + grp * _L) * _NW
        t0 = bvec * _BAND
        sloc_v[pl.ds(grp * _L, _L)] = bsearch(t0)
        eloc_v[pl.ds(grp * _L, _L)] = bsearch(t0 + _BAND)

    def clamped_band(k):
        return jnp.minimum(wid + k * _NW, _NFULL - 1)

    def fire_band(k, par):
        b0 = pl.multiple_of(clamped_band(k) * _BAND, _BAND)
        for tc in range(8):
            pltpu.async_copy(
                entt_hbm.at[pl.ds(tc * 8, 8), pl.ds(b0, _BAND)],
                band_v.at[pl.ds(par * _D + tc * 8, 8)], sems[par])

    def wait_band(par):
        pltpu.make_async_copy(entt_hbm.at[pl.ds(0, _D), pl.ds(0, _BAND)],
                              band_v.at[pl.ds(par * _D, _D)],
                              sems[par]).wait()

    def splat(x):
        return jnp.full((_L,), x, jnp.int32)

    def emit_positions(s, e, cnt, src_ref, row_off, tail_mode):
        def pos_body(p, cnt):
            e_id = plsc.load_gather(ids_v, [splat(p)])[0]
            pm = plsc.load_gather(perm_v, [splat(p)])[0]
            col = splat(e_id - _TAIL0 if tail_mode else e_id & (_BAND - 1))
            is_h = pm < _B
            slot = cnt % _RING
            for kk in range(4):
                v = plsc.load_gather(src_ref,
                                     [row_off + lanes + kk * _L, col])
                v = jnp.where(is_h, v + rel_c[kk], v)
                ring_v[pl.ds(slot * _D + kk * _L, _L)] = v
            pltpu.async_copy(ring_v.at[pl.ds(slot * _D, _D)],
                             rows_hbm.at[pl.ds(pm * _D, _D)], semr)
            cnt = cnt + 1

            @pl.when(cnt % _RING == 0)
            def _():
                pltpu.make_async_copy(rows_hbm.at[pl.ds(0, _RING * _D)],
                                      ring_v, semr).wait()

            return cnt

        return lax.fori_loop(s, e, pos_body, cnt)

    def process_band(k, cnt, par):
        s = plsc.load_gather(sloc_v, [splat(k)])[0]
        e = plsc.load_gather(eloc_v, [splat(k)])[0]
        return emit_positions(s, e, cnt, band_v, par * _D, False)

    for k in range(3):
        fire_band(k, k)

    def quad_body(k4, cnt):
        for par in range(4):
            k = 4 * k4 + par
            wait_band(par)
            fire_band(k + 3, (par + 3) % 4)
            cnt = process_band(k, cnt, par)
        return cnt

    cnt = lax.fori_loop(0, nb4, quad_body, jnp.int32(0))
    for par in range(3):
        wait_band(par)

    tail_s = jnp.where(wid == _NW - 1, bsearch(splat(_TAIL0))[0],
                       jnp.int32(_NIDS))
    cnt = emit_positions(tail_s, jnp.int32(_NIDS), cnt, tail_v, 0, True)

    def drain_body(i, x):
        pltpu.make_async_copy(rows_hbm.at[pl.ds(0, _D)],
                              ring_v.at[pl.ds(0, _D)], semr).wait()
        return x

    lax.fori_loop(0, cnt % _RING, drain_body, 0)


_sc_gather = pl.kernel(
    _gather_body,
    out_type=jax.ShapeDtypeStruct((_NIDS * _D,), jnp.float32),
    mesh=_mesh,
    compiler_params=_params,
    scratch_types=[
        pltpu.VMEM((_NIDS,), jnp.int32),
        pltpu.VMEM((_NIDS,), jnp.int32),
        pltpu.VMEM((256,), jnp.int32),
        pltpu.VMEM((256,), jnp.int32),
        pltpu.VMEM((4 * _D, _BAND), jnp.float32),
        pltpu.VMEM((_D, _D), jnp.float32),
        pltpu.VMEM((_D,), jnp.float32),
        pltpu.VMEM((_RING * _D,), jnp.float32),
        pltpu.SemaphoreType.DMA,
        pltpu.SemaphoreType.DMA,
        pltpu.SemaphoreType.DMA,
        pltpu.SemaphoreType.DMA,
        pltpu.SemaphoreType.DMA,
    ],
)


def _score_body(rows_hbm, out_hbm, hrow_v, trow_v, out_v, sem0, sem1):
    wid = lax.axis_index("s") * _NC + lax.axis_index("c")
    base = wid * _BPW
    lanes = lax.iota(jnp.int32, _L)
    for c in range(_NCH):
        off = base + c * _CH
        cp_h = pltpu.async_copy(rows_hbm.at[pl.ds(off * _D, _CH * _D)],
                                hrow_v, sem0)
        cp_t = pltpu.async_copy(
            rows_hbm.at[pl.ds((_B + off) * _D, _CH * _D)], trow_v, sem1)
        cp_h.wait()
        cp_t.wait()

        def group_body(g, carry):
            rows = (lanes + g * _L) * _D

            def dim_body(d, acc):
                idx = rows + d
                hd = plsc.load_gather(hrow_v, [idx])
                td = plsc.load_gather(trow_v, [idx])
                diff = hd - td
                return acc + diff * diff

            acc = lax.fori_loop(0, _D, dim_body,
                                jnp.zeros((_L,), jnp.float32))
            out_v[pl.ds(c * _CH + g * _L, _L)] = _vsqrt(acc + 1e-12)
            return carry

        lax.fori_loop(0, _CH // _L, group_body, 0)
    pltpu.sync_copy(out_v, out_hbm.at[pl.ds(base, _BPW)])


_sc_score = pl.kernel(
    _score_body,
    out_type=jax.ShapeDtypeStruct((_B,), jnp.float32),
    mesh=_mesh,
    compiler_params=_params,
    scratch_types=[
        pltpu.VMEM((_CH * _D,), jnp.float32),
        pltpu.VMEM((_CH * _D,), jnp.float32),
        pltpu.VMEM((_BPW,), jnp.float32),
        pltpu.SemaphoreType.DMA,
        pltpu.SemaphoreType.DMA,
    ],
)


@jax.jit
def kernel(data, node_ids, entity_emb, relation_emb):
    hidx = data[:, 0]
    tidx = data[:, 1]
    ids = _sc_remap(hidx, tidx, node_ids)
    iota = lax.iota(jnp.int32, _NIDS)
    sids, perm = lax.sort_key_val(ids, iota)
    entt = entity_emb.T
    tail = entt[:, _TAIL0:]
    rel_row = relation_emb[_REL]
    rows = _sc_gather(sids, perm, entt, tail, rel_row)
    out = _sc_score(rows)
    return out[:, None]


# speedup vs baseline: 4.4949x; 1.0146x over previous
import jax
import jax.numpy as jnp
from jax import lax
from jax.experimental import pallas as pl
from jax.experimental.pallas import tpu as pltpu
from jax.experimental.pallas import tpu_sc as plsc

_B = 16384
_D = 64
_REL = 5
_L = 16
_NC = 2
_NS = 16
_NW = _NC * _NS
_BPW = _B // _NW
_CH = 128
_NCH = _BPW // _CH
_NE = 1000000
_BAND = 128
_NFULL = _NE // _BAND
_TAIL0 = _NFULL * _BAND
_NIDS = 2 * _B
_NST = _NFULL + 2
_RING = 32

_mesh = plsc.VectorSubcoreMesh(core_axis_name="c", subcore_axis_name="s",
                               num_cores=_NC, num_subcores=_NS)
_params = pltpu.CompilerParams(needs_layout_passes=False)


def _vsqrt(x):
    i = plsc.bitcast(x, jnp.int32)
    i = jnp.int32(0x5F3759DF) - lax.shift_right_logical(i, 1)
    y = plsc.bitcast(i, jnp.float32)
    for _ in range(3):
        y = y * (1.5 - 0.5 * x * y * y)
    return x * y


def _remap_body(hidx_hbm, tidx_hbm, nid_hbm, ids_hbm,
                hi_v, ti_v, hm_v, tm_v, sem0, sem1):
    wid = lax.axis_index("s") * _NC + lax.axis_index("c")
    base = wid * _BPW
    for c in range(_NCH):
        off = base + c * _CH
        pltpu.sync_copy(hidx_hbm.at[pl.ds(off, _CH)], hi_v)
        pltpu.sync_copy(tidx_hbm.at[pl.ds(off, _CH)], ti_v)
        cp_h = pltpu.async_copy(nid_hbm.at[hi_v], hm_v, sem0)
        cp_t = pltpu.async_copy(nid_hbm.at[ti_v], tm_v, sem1)
        cp_h.wait()
        cp_t.wait()
        pltpu.sync_copy(hm_v, ids_hbm.at[pl.ds(off, _CH)])
        pltpu.sync_copy(tm_v, ids_hbm.at[pl.ds(_B + off, _CH)])


_sc_remap = pl.kernel(
    _remap_body,
    out_type=jax.ShapeDtypeStruct((_NIDS,), jnp.int32),
    mesh=_mesh,
    compiler_params=_params,
    scratch_types=[
        pltpu.VMEM((_CH,), jnp.int32),
        pltpu.VMEM((_CH,), jnp.int32),
        pltpu.VMEM((_CH,), jnp.int32),
        pltpu.VMEM((_CH,), jnp.int32),
        pltpu.SemaphoreType.DMA,
        pltpu.SemaphoreType.DMA,
    ],
)


def _gather_body(sids_hbm, perm_hbm, entt_hbm, tail_hbm, rel_hbm,
                 rows_hbm, ids_v, perm_v, sloc_v, eloc_v, band_v, tail_v,
                 rel_v, ring_v, semb0, semb1, semb2, semb3, semr):
    wid = lax.axis_index("s") * _NC + lax.axis_index("c")
    pltpu.sync_copy(sids_hbm, ids_v)
    pltpu.sync_copy(perm_hbm, perm_v)
    pltpu.sync_copy(tail_hbm, tail_v)
    pltpu.sync_copy(rel_hbm, rel_v)
    lanes = lax.iota(jnp.int32, _L)
    rel_c = [rel_v[pl.ds(kk * _L, _L)] for kk in range(4)]
    sems = [semb0, semb1, semb2, semb3]
    nb = (_NFULL - 1 - wid) // _NW + 1
    nb4 = (nb + 3) // 4

    def bsearch(tgt):
        lo = jnp.zeros((_L,), jnp.int32)
        hi = jnp.full((_L,), _NIDS, jnp.int32)
        for _ in range(16):
            act = lo < hi
            mid = lax.shift_right_logical(lo + hi, 1)
            v = plsc.load_gather(ids_v,
                                 [jnp.minimum(mid, _NIDS - 1)])
            right = jnp.logical_and(act, v < tgt)
            lo = jnp.where(right, mid + 1, lo)
            hi = jnp.where(jnp.logical_and(act, jnp.logical_not(right)),
                           mid, hi)
        return lo

    for grp in range(16):
        bvec = wid + (lanes + grp * _L) * _NW
        t0 = bvec * _BAND
        sloc_v[pl.ds(grp * _L, _L)] = bsearch(t0)
        eloc_v[pl.ds(grp * _L, _L)] = bsearch(t0 + _BAND)

    def clamped_band(k):
        return jnp.minimum(wid + k * _NW, _NFULL - 1)

    def fire_band(k, par):
        b0 = pl.multiple_of(clamped_band(k) * _BAND, _BAND)
        for tc in range(8):
            pltpu.async_copy(
                entt_hbm.at[pl.ds(tc * 8, 8), pl.ds(b0, _BAND)],
                band_v.at[pl.ds(par * _D + tc * 8, 8)], sems[par])

    def wait_band(par):
        pltpu.make_async_copy(entt_hbm.at[pl.ds(0, _D), pl.ds(0, _BAND)],
                              band_v.at[pl.ds(par * _D, _D)],
                              sems[par]).wait()

    def splat(x):
        return jnp.full((_L,), x, jnp.int32)

    def emit_positions(s, e, cnt, src_ref, row_off, tail_mode):
        def pos_body(p, cnt):
            e_id = plsc.load_gather(ids_v, [splat(p)])[0]
            pm = plsc.load_gather(perm_v, [splat(p)])[0]
            col = splat(e_id - _TAIL0 if tail_mode else e_id & (_BAND - 1))
            is_h = pm < _B
            slot = cnt % _RING
            for kk in range(4):
                v = plsc.load_gather(src_ref,
                                     [row_off + lanes + kk * _L, col])
                v = jnp.where(is_h, v + rel_c[kk], v)
                ring_v[pl.ds(slot * _D + kk * _L, _L)] = v
            pltpu.async_copy(ring_v.at[pl.ds(slot * _D, _D)],
                             rows_hbm.at[pl.ds(pm * _D, _D)], semr)
            cnt = cnt + 1

            @pl.when(cnt % _RING == 0)
            def _():
                pltpu.make_async_copy(rows_hbm.at[pl.ds(0, _RING * _D)],
                                      ring_v, semr).wait()

            return cnt

        return lax.fori_loop(s, e, pos_body, cnt)

    def process_band(k, cnt, par):
        s = plsc.load_gather(sloc_v, [splat(k)])[0]
        e = plsc.load_gather(eloc_v, [splat(k)])[0]
        return emit_positions(s, e, cnt, band_v, par * _D, False)

    for k in range(3):
        fire_band(k, k)

    def quad_body(k4, cnt):
        for par in range(4):
            k = 4 * k4 + par
            wait_band(par)
            fire_band(k + 3, (par + 3) % 4)
            cnt = process_band(k, cnt, par)
        return cnt

    cnt = lax.fori_loop(0, nb4, quad_body, jnp.int32(0))
    for par in range(3):
        wait_band(par)

    tail_s = jnp.where(wid == _NW - 1, bsearch(splat(_TAIL0))[0],
                       jnp.int32(_NIDS))
    cnt = emit_positions(tail_s, jnp.int32(_NIDS), cnt, tail_v, 0, True)

    def drain_body(i, x):
        pltpu.make_async_copy(rows_hbm.at[pl.ds(0, _D)],
                              ring_v.at[pl.ds(0, _D)], semr).wait()
        return x

    lax.fori_loop(0, cnt % _RING, drain_body, 0)


_sc_gather = pl.kernel(
    _gather_body,
    out_type=jax.ShapeDtypeStruct((_NIDS * _D,), jnp.float32),
    mesh=_mesh,
    compiler_params=_params,
    scratch_types=[
        pltpu.VMEM((_NIDS,), jnp.int32),
        pltpu.VMEM((_NIDS,), jnp.int32),
        pltpu.VMEM((256,), jnp.int32),
        pltpu.VMEM((256,), jnp.int32),
        pltpu.VMEM((4 * _D, _BAND), jnp.float32),
        pltpu.VMEM((_D, _D), jnp.float32),
        pltpu.VMEM((_D,), jnp.float32),
        pltpu.VMEM((_RING * _D,), jnp.float32),
        pltpu.SemaphoreType.DMA,
        pltpu.SemaphoreType.DMA,
        pltpu.SemaphoreType.DMA,
        pltpu.SemaphoreType.DMA,
        pltpu.SemaphoreType.DMA,
    ],
)


def _score_body(rows_hbm, out_hbm, hrow_v, trow_v, out_v, semh0, semh1,
                semt0, semt1):
    wid = lax.axis_index("s") * _NC + lax.axis_index("c")
    base = wid * _BPW
    lanes = lax.iota(jnp.int32, _L)
    semh = [semh0, semh1]
    semt = [semt0, semt1]

    def fire(c, par):
        off = base + c * _CH
        pltpu.async_copy(rows_hbm.at[pl.ds(off * _D, _CH * _D)],
                         hrow_v.at[pl.ds(par * _CH * _D, _CH * _D)],
                         semh[par])
        pltpu.async_copy(rows_hbm.at[pl.ds((_B + off) * _D, _CH * _D)],
                         trow_v.at[pl.ds(par * _CH * _D, _CH * _D)],
                         semt[par])

    def wait(par):
        pltpu.make_async_copy(rows_hbm.at[pl.ds(0, _CH * _D)],
                              hrow_v.at[pl.ds(par * _CH * _D, _CH * _D)],
                              semh[par]).wait()
        pltpu.make_async_copy(rows_hbm.at[pl.ds(0, _CH * _D)],
                              trow_v.at[pl.ds(par * _CH * _D, _CH * _D)],
                              semt[par]).wait()

    fire(0, 0)
    for c in range(_NCH):
        par = c % 2
        wait(par)
        if c + 1 < _NCH:
            fire(c + 1, 1 - par)
        vbase = par * _CH * _D

        def group_body(g, carry):
            rows = vbase + (lanes + g * _L) * _D

            def dim_body(j, acc):
                for dd in range(4):
                    idx = rows + j * 4 + dd
                    hd = plsc.load_gather(hrow_v, [idx])
                    td = plsc.load_gather(trow_v, [idx])
                    diff = hd - td
                    acc = acc + diff * diff
                return acc

            acc = lax.fori_loop(0, _D // 4, dim_body,
                                jnp.zeros((_L,), jnp.float32))
            out_v[pl.ds(c * _CH + g * _L, _L)] = _vsqrt(acc + 1e-12)
            return carry

        lax.fori_loop(0, _CH // _L, group_body, 0)
    pltpu.sync_copy(out_v, out_hbm.at[pl.ds(base, _BPW)])


_sc_score = pl.kernel(
    _score_body,
    out_type=jax.ShapeDtypeStruct((_B,), jnp.float32),
    mesh=_mesh,
    compiler_params=_params,
    scratch_types=[
        pltpu.VMEM((2 * _CH * _D,), jnp.float32),
        pltpu.VMEM((2 * _CH * _D,), jnp.float32),
        pltpu.VMEM((_BPW,), jnp.float32),
        pltpu.SemaphoreType.DMA,
        pltpu.SemaphoreType.DMA,
        pltpu.SemaphoreType.DMA,
        pltpu.SemaphoreType.DMA,
    ],
)


@jax.jit
def kernel(data, node_ids, entity_emb, relation_emb):
    hidx = data[:, 0]
    tidx = data[:, 1]
    ids = _sc_remap(hidx, tidx, node_ids)
    iota = lax.iota(jnp.int32, _NIDS)
    sids, perm = lax.sort_key_val(ids, iota)
    entt = entity_emb.T
    tail = entt[:, _TAIL0:]
    rel_row = relation_emb[_REL]
    rows = _sc_gather(sids, perm, entt, tail, rel_row)
    out = _sc_score(rows)
    return out[:, None]


# speedup vs baseline: 4.5253x; 1.0068x over previous
import jax
import jax.numpy as jnp
from jax import lax
from jax.experimental import pallas as pl
from jax.experimental.pallas import tpu as pltpu
from jax.experimental.pallas import tpu_sc as plsc

_B = 16384
_D = 64
_REL = 5
_L = 16
_NC = 2
_NS = 16
_NW = _NC * _NS
_BPW = _B // _NW
_CH = 128
_NCH = _BPW // _CH
_NE = 1000000
_BAND = 128
_NFULL = _NE // _BAND
_TAIL0 = _NFULL * _BAND
_NIDS = 2 * _B
_NST = _NFULL + 2
_RING = 32

_mesh = plsc.VectorSubcoreMesh(core_axis_name="c", subcore_axis_name="s",
                               num_cores=_NC, num_subcores=_NS)
_params = pltpu.CompilerParams(needs_layout_passes=False)


def _vsqrt(x):
    i = plsc.bitcast(x, jnp.int32)
    i = jnp.int32(0x5F3759DF) - lax.shift_right_logical(i, 1)
    y = plsc.bitcast(i, jnp.float32)
    for _ in range(3):
        y = y * (1.5 - 0.5 * x * y * y)
    return x * y


def _remap_body(hidx_hbm, tidx_hbm, nid_hbm, ids_hbm,
                hi_v, ti_v, hm_v, tm_v, sem0, sem1):
    wid = lax.axis_index("s") * _NC + lax.axis_index("c")
    base = wid * _BPW
    for c in range(_NCH):
        off = base + c * _CH
        pltpu.sync_copy(hidx_hbm.at[pl.ds(off, _CH)], hi_v)
        pltpu.sync_copy(tidx_hbm.at[pl.ds(off, _CH)], ti_v)
        cp_h = pltpu.async_copy(nid_hbm.at[hi_v], hm_v, sem0)
        cp_t = pltpu.async_copy(nid_hbm.at[ti_v], tm_v, sem1)
        cp_h.wait()
        cp_t.wait()
        pltpu.sync_copy(hm_v, ids_hbm.at[pl.ds(off, _CH)])
        pltpu.sync_copy(tm_v, ids_hbm.at[pl.ds(_B + off, _CH)])


_sc_remap = pl.kernel(
    _remap_body,
    out_type=jax.ShapeDtypeStruct((_NIDS,), jnp.int32),
    mesh=_mesh,
    compiler_params=_params,
    scratch_types=[
        pltpu.VMEM((_CH,), jnp.int32),
        pltpu.VMEM((_CH,), jnp.int32),
        pltpu.VMEM((_CH,), jnp.int32),
        pltpu.VMEM((_CH,), jnp.int32),
        pltpu.SemaphoreType.DMA,
        pltpu.SemaphoreType.DMA,
    ],
)


def _gather_body(sids_hbm, perm_hbm, entt_hbm, tail_hbm, rel_hbm,
                 rows_hbm, ids_v, perm_v, sloc_v, eloc_v, band_v, tail_v,
                 rel_v, ring_v, semb0, semb1, semb2, semb3, semr):
    wid = lax.axis_index("s") * _NC + lax.axis_index("c")
    pltpu.sync_copy(sids_hbm, ids_v)
    pltpu.sync_copy(perm_hbm, perm_v)
    pltpu.sync_copy(tail_hbm, tail_v)
    pltpu.sync_copy(rel_hbm, rel_v)
    lanes = lax.iota(jnp.int32, _L)
    rel_c = [rel_v[pl.ds(kk * _L, _L)] for kk in range(4)]
    sems = [semb0, semb1, semb2, semb3]
    nb = (_NFULL - 1 - wid) // _NW + 1
    nb4 = (nb + 3) // 4

    def bsearch(tgt):
        lo = jnp.zeros((_L,), jnp.int32)
        hi = jnp.full((_L,), _NIDS, jnp.int32)
        for _ in range(16):
            act = lo < hi
            mid = lax.shift_right_logical(lo + hi, 1)
            v = plsc.load_gather(ids_v,
                                 [jnp.minimum(mid, _NIDS - 1)])
            right = jnp.logical_and(act, v < tgt)
            lo = jnp.where(right, mid + 1, lo)
            hi = jnp.where(jnp.logical_and(act, jnp.logical_not(right)),
                           mid, hi)
        return lo

    for grp in range(16):
        bvec = wid + (lanes + grp * _L) * _NW
        t0 = bvec * _BAND
        sloc_v[pl.ds(grp * _L, _L)] = bsearch(t0)
        eloc_v[pl.ds(grp * _L, _L)] = bsearch(t0 + _BAND)

    def clamped_band(k):
        return jnp.minimum(wid + k * _NW, _NFULL - 1)

    def fire_band(k, par):
        b0 = pl.multiple_of(clamped_band(k) * _BAND, _BAND)
        for tc in range(8):
            pltpu.async_copy(
                entt_hbm.at[pl.ds(tc * 8, 8), pl.ds(b0, _BAND)],
                band_v.at[pl.ds(par * _D + tc * 8, 8)], sems[par])

    def wait_band(par):
        pltpu.make_async_copy(entt_hbm.at[pl.ds(0, _D), pl.ds(0, _BAND)],
                              band_v.at[pl.ds(par * _D, _D)],
                              sems[par]).wait()

    def splat(x):
        return jnp.full((_L,), x, jnp.int32)

    def emit_positions(s, e, cnt, src_ref, row_off, tail_mode):
        def pos_body(p, cnt):
            e_id = plsc.load_gather(ids_v, [splat(p)])[0]
            pm = plsc.load_gather(perm_v, [splat(p)])[0]
            col = splat(e_id - _TAIL0 if tail_mode else e_id & (_BAND - 1))
            is_h = pm < _B
            slot = cnt % _RING
            for kk in range(4):
                v = plsc.load_gather(src_ref,
                                     [row_off + lanes + kk * _L, col])
                v = jnp.where(is_h, v + rel_c[kk], v)
                ring_v[pl.ds(slot * _D + kk * _L, _L)] = v
            pltpu.async_copy(ring_v.at[pl.ds(slot * _D, _D)],
                             rows_hbm.at[pl.ds(pm * _D, _D)], semr)
            cnt = cnt + 1

            @pl.when(cnt % _RING == 0)
            def _():
                pltpu.make_async_copy(rows_hbm.at[pl.ds(0, _RING * _D)],
                                      ring_v, semr).wait()

            return cnt

        return lax.fori_loop(s, e, pos_body, cnt)

    def process_band(k, cnt, par):
        s = plsc.load_gather(sloc_v, [splat(k)])[0]
        e = plsc.load_gather(eloc_v, [splat(k)])[0]
        return emit_positions(s, e, cnt, band_v, par * _D, False)

    for k in range(3):
        fire_band(k, k)

    def quad_body(k4, cnt):
        for par in range(4):
            k = 4 * k4 + par
            wait_band(par)
            fire_band(k + 3, (par + 3) % 4)
            cnt = process_band(k, cnt, par)
        return cnt

    cnt = lax.fori_loop(0, nb4, quad_body, jnp.int32(0))
    for par in range(3):
        wait_band(par)

    tail_s = jnp.where(wid == _NW - 1, bsearch(splat(_TAIL0))[0],
                       jnp.int32(_NIDS))
    cnt = emit_positions(tail_s, jnp.int32(_NIDS), cnt, tail_v, 0, True)

    def drain_body(i, x):
        pltpu.make_async_copy(rows_hbm.at[pl.ds(0, _D)],
                              ring_v.at[pl.ds(0, _D)], semr).wait()
        return x

    lax.fori_loop(0, cnt % _RING, drain_body, 0)


_sc_gather = pl.kernel(
    _gather_body,
    out_type=jax.ShapeDtypeStruct((_NIDS * _D,), jnp.float32),
    mesh=_mesh,
    compiler_params=_params,
    scratch_types=[
        pltpu.VMEM((_NIDS,), jnp.int32),
        pltpu.VMEM((_NIDS,), jnp.int32),
        pltpu.VMEM((256,), jnp.int32),
        pltpu.VMEM((256,), jnp.int32),
        pltpu.VMEM((4 * _D, _BAND), jnp.float32),
        pltpu.VMEM((_D, _D), jnp.float32),
        pltpu.VMEM((_D,), jnp.float32),
        pltpu.VMEM((_RING * _D,), jnp.float32),
        pltpu.SemaphoreType.DMA,
        pltpu.SemaphoreType.DMA,
        pltpu.SemaphoreType.DMA,
        pltpu.SemaphoreType.DMA,
        pltpu.SemaphoreType.DMA,
    ],
)


def _score_body(rows_hbm, out_hbm, hrow_v, trow_v, out_v, semh0, semh1,
                semt0, semt1):
    wid = lax.axis_index("s") * _NC + lax.axis_index("c")
    base = wid * _BPW
    lanes = lax.iota(jnp.int32, _L)
    semh = [semh0, semh1]
    semt = [semt0, semt1]

    def fire(c, par):
        off = base + c * _CH
        pltpu.async_copy(rows_hbm.at[pl.ds(off * _D, _CH * _D)],
                         hrow_v.at[pl.ds(par * _CH * _D, _CH * _D)],
                         semh[par])
        pltpu.async_copy(rows_hbm.at[pl.ds((_B + off) * _D, _CH * _D)],
                         trow_v.at[pl.ds(par * _CH * _D, _CH * _D)],
                         semt[par])

    def wait(par):
        pltpu.make_async_copy(rows_hbm.at[pl.ds(0, _CH * _D)],
                              hrow_v.at[pl.ds(par * _CH * _D, _CH * _D)],
                              semh[par]).wait()
        pltpu.make_async_copy(rows_hbm.at[pl.ds(0, _CH * _D)],
                              trow_v.at[pl.ds(par * _CH * _D, _CH * _D)],
                              semt[par]).wait()

    fire(0, 0)
    for c in range(_NCH):
        par = c % 2
        wait(par)
        if c + 1 < _NCH:
            fire(c + 1, 1 - par)
        vbase = par * _CH * _D

        def group_body(g, carry):
            rows = vbase + (lanes + g * _L) * _D
            zero = jnp.zeros((_L,), jnp.float32)

            def dim_body(j, accs):
                out = []
                for dd in range(4):
                    idx = rows + j * 4 + dd
                    hd = plsc.load_gather(hrow_v, [idx])
                    td = plsc.load_gather(trow_v, [idx])
                    diff = hd - td
                    out.append(accs[dd] + diff * diff)
                return tuple(out)

            accs = lax.fori_loop(0, _D // 4, dim_body,
                                 (zero, zero, zero, zero))
            acc = (accs[0] + accs[1]) + (accs[2] + accs[3])
            out_v[pl.ds(c * _CH + g * _L, _L)] = _vsqrt(acc + 1e-12)
            return carry

        lax.fori_loop(0, _CH // _L, group_body, 0)
    pltpu.sync_copy(out_v, out_hbm.at[pl.ds(base, _BPW)])


_sc_score = pl.kernel(
    _score_body,
    out_type=jax.ShapeDtypeStruct((_B,), jnp.float32),
    mesh=_mesh,
    compiler_params=_params,
    scratch_types=[
        pltpu.VMEM((2 * _CH * _D,), jnp.float32),
        pltpu.VMEM((2 * _CH * _D,), jnp.float32),
        pltpu.VMEM((_BPW,), jnp.float32),
        pltpu.SemaphoreType.DMA,
        pltpu.SemaphoreType.DMA,
        pltpu.SemaphoreType.DMA,
        pltpu.SemaphoreType.DMA,
    ],
)


@jax.jit
def kernel(data, node_ids, entity_emb, relation_emb):
    hidx = data[:, 0]
    tidx = data[:, 1]
    ids = _sc_remap(hidx, tidx, node_ids)
    iota = lax.iota(jnp.int32, _NIDS)
    sids, perm = lax.sort_key_val(ids, iota)
    entt = entity_emb.T
    tail = entt[:, _TAIL0:]
    rel_row = relation_emb[_REL]
    rows = _sc_gather(sids, perm, entt, tail, rel_row)
    out = _sc_score(rows)
    return out[:, None]
